# baseline (speedup 1.0000x reference)
import functools

import jax
import jax.numpy as jnp
from jax import lax
from jax.experimental import pallas as pl
from jax.experimental.pallas import tpu as pltpu
from jax.experimental.pallas import tpu_sc as plsc

N_GRAPH, N_NODE, N_FEAT = 256, 128, 9
HIDDEN = 768
W2 = HIDDEN // 2
NUM_ATOMS_P1 = 4609
NUM_IN_DEG = 512
NUM_OUT_DEG = 512

NW = 32
GPW = N_GRAPH // NW
NODES_PW = GPW * N_NODE
K = N_FEAT + 2
C = 8
AROWS = C * N_FEAT
DROWS = C
ROWS_PER_STEP = AROWS + 2 * DROWS
STEPS = NODES_PW // C
STEPS_PER_GRAPH = N_NODE // C
OUT_ROW_STRIDE = (N_NODE + 1) * HIDDEN
LANES = 16
NWRD = W2 // LANES
GUNROLL = 2
HIMASK = jnp.int32(-65536)


def _out_base(wid, s):
    g = wid * GPW + s // STEPS_PER_GRAPH
    n0 = (s % STEPS_PER_GRAPH) * C
    return g * OUT_ROW_STRIDE + (1 + n0) * HIDDEN


def _body(atab_hbm, itab_hbm, otab_hbm, aidx_hbm, iidx_hbm, oidx_hbm,
          token_hbm, out_hbm,
          aidx_v, iidx_v, oidx_v, gb0, gb1, ab0, ab1, token_v,
          sg0, sg1, so0, so1):
    wid = lax.axis_index("s") * 2 + lax.axis_index("c")
    g0 = wid * GPW
    gbufs = (gb0, gb1)
    accbs = (ab0, ab1)
    sgs = (sg0, sg1)
    sos = (so0, so1)

    pltpu.sync_copy(aidx_hbm.at[pl.ds(wid * NODES_PW * N_FEAT, NODES_PW * N_FEAT)],
                    aidx_v)
    pltpu.sync_copy(iidx_hbm.at[pl.ds(wid * NODES_PW, NODES_PW)], iidx_v)
    pltpu.sync_copy(oidx_hbm.at[pl.ds(wid * NODES_PW, NODES_PW)], oidx_v)

    pltpu.sync_copy(token_hbm, token_v)
    for g in range(GPW):
        pltpu.sync_copy(token_v, out_hbm.at[pl.ds((g0 + g) * OUT_ROW_STRIDE, HIDDEN)])

    def gather(s, p):
        pltpu.async_copy(
            atab_hbm.at[aidx_v.at[pl.ds(s * AROWS, AROWS)]],
            gbufs[p].at[pl.ds(0, AROWS)], sgs[p])
        pltpu.async_copy(
            itab_hbm.at[iidx_v.at[pl.ds(s * DROWS, DROWS)]],
            gbufs[p].at[pl.ds(AROWS, DROWS)], sgs[p])
        pltpu.async_copy(
            otab_hbm.at[oidx_v.at[pl.ds(s * DROWS, DROWS)]],
            gbufs[p].at[pl.ds(AROWS + DROWS, DROWS)], sgs[p])

    gather(0, 0)

    def pair(s2, carry):
        for p in (0, 1):
            s = 2 * s2 + p
            q = 1 - p
            @pl.when(s + 1 < STEPS)
            def _():
                gather(s + 1, q)

            pltpu.make_async_copy(atab_hbm.at[pl.ds(0, ROWS_PER_STEP)],
                                  gbufs[p], sgs[p]).wait()

            @pl.when(s2 >= 1)
            def _():
                pltpu.make_async_copy(accbs[p], out_hbm.at[pl.ds(0, C * HIDDEN)],
                                      sos[p]).wait()

            gb = gbufs[p]
            ab = accbs[p]
            for j in range(C):
                def wordg(gg, _, j=j):
                    for u in range(GUNROLL):
                        g = gg * GUNROLL + u
                        gs = pl.ds(g * LANES, LANES)
                        w = gb[j * N_FEAT, gs]
                        a = plsc.bitcast(w << 16, jnp.float32)
                        b = plsc.bitcast(w & HIMASK, jnp.float32)
                        for t in range(1, N_FEAT):
                            w = gb[j * N_FEAT + t, gs]
                            a = a + plsc.bitcast(w << 16, jnp.float32)
                            b = b + plsc.bitcast(w & HIMASK, jnp.float32)
                        for r in (AROWS + j, AROWS + DROWS + j):
                            w = gb[r, gs]
                            a = a + plsc.bitcast(w << 16, jnp.float32)
                            b = b + plsc.bitcast(w & HIMASK, jnp.float32)
                        ab[pl.ds(j * HIDDEN + g * LANES, LANES)] = a
                        ab[pl.ds(j * HIDDEN + W2 + g * LANES, LANES)] = b
                    return 0

                lax.fori_loop(0, NWRD // GUNROLL, wordg, 0)

            pltpu.async_copy(ab, out_hbm.at[pl.ds(_out_base(wid, s), C * HIDDEN)],
                             sos[p])
        return carry

    lax.fori_loop(0, STEPS // 2, pair, 0)

    for p in (0, 1):
        pltpu.make_async_copy(accbs[p], out_hbm.at[pl.ds(0, C * HIDDEN)],
                              sos[p]).wait()


def _sc_lookup(atab, itab, otab, aidx, iidx, oidx, graph_token):
    mesh = plsc.VectorSubcoreMesh(core_axis_name="c", subcore_axis_name="s")
    fn = functools.partial(
        pl.kernel,
        mesh=mesh,
        compiler_params=pltpu.CompilerParams(needs_layout_passes=False),
        out_type=jax.ShapeDtypeStruct((N_GRAPH * (N_NODE + 1) * HIDDEN,), jnp.float32),
        scratch_types=[
            pltpu.VMEM((NODES_PW * N_FEAT,), jnp.int32),
            pltpu.VMEM((NODES_PW,), jnp.int32),
            pltpu.VMEM((NODES_PW,), jnp.int32),
            pltpu.VMEM((ROWS_PER_STEP, W2), jnp.int32),
            pltpu.VMEM((ROWS_PER_STEP, W2), jnp.int32),
            pltpu.VMEM((C * HIDDEN,), jnp.float32),
            pltpu.VMEM((C * HIDDEN,), jnp.float32),
            pltpu.VMEM((HIDDEN,), jnp.float32),
            pltpu.SemaphoreType.DMA,
            pltpu.SemaphoreType.DMA,
            pltpu.SemaphoreType.DMA,
            pltpu.SemaphoreType.DMA,
        ],
    )(_body)
    return fn(atab, itab, otab, aidx, iidx, oidx, graph_token.reshape(HIDDEN))


def _pack_body(x_ref, o_ref):
    x = x_ref[...]
    lo = jax.lax.bitcast_convert_type(
        x[:, :W2].astype(jnp.bfloat16), jnp.uint16).astype(jnp.uint32)
    hi = jax.lax.bitcast_convert_type(
        x[:, W2:].astype(jnp.bfloat16), jnp.uint16).astype(jnp.uint32)
    o_ref[...] = jax.lax.bitcast_convert_type(lo | (hi << 16), jnp.int32)


def _pack(table):
    v = table.shape[0]
    blocks = (v + 7) // 8
    return pl.pallas_call(
        _pack_body,
        grid=(blocks,),
        in_specs=[pl.BlockSpec((8, HIDDEN), lambda i: (i, 0))],
        out_specs=pl.BlockSpec((8, W2), lambda i: (i, 0)),
        out_shape=jax.ShapeDtypeStruct((v, W2), jnp.int32),
    )(table)


def kernel(input_nodes, in_degree, out_degree, atom_table, in_deg_table,
           out_deg_table, graph_token):
    atab = _pack(atom_table)
    itab = _pack(in_deg_table)
    otab = _pack(out_deg_table)
    aidx = input_nodes.astype(jnp.int32).reshape(-1)
    iidx = in_degree.astype(jnp.int32).reshape(-1)
    oidx = out_degree.astype(jnp.int32).reshape(-1)
    flat = _sc_lookup(atab, itab, otab, aidx, iidx, oidx, graph_token)
    return flat.reshape(N_GRAPH, N_NODE + 1, HIDDEN)
